# baseline (speedup 1.0000x reference)
import functools

import jax
import jax.numpy as jnp
from jax import lax
from jax.experimental import pallas as pl
from jax.experimental.pallas import tpu as pltpu
from jax.experimental.pallas import tpu_sc as plsc

FEAT = 1024
KDIM = 1024
VDIM = 1024
DICT = 2048
NQ = 16384
K = 5
KPAD = 16
TEMP = 0.15
BN = 256

NWORK = 32
QPW = NQ // NWORK
CH = 16
NCHUNK = QPW // CH


def _norm_kernel(keys_ref, kn_ref):
    keys = keys_ref[...]
    kn_ref[...] = keys / jnp.maximum(
        jnp.sqrt(jnp.sum(keys * keys, axis=1, keepdims=True)), 1e-12)


def _block_kernel(q_ref, w1_ref, b1_ref, w2_ref, b2_ref, kn_ref,
                  sim_ref, idx_ref, w_ref):
    x = q_ref[...]
    h = jnp.maximum(
        jnp.dot(x, w1_ref[...], preferred_element_type=jnp.float32)
        + b1_ref[...], 0.0)
    qp = jnp.dot(h, w2_ref[...], preferred_element_type=jnp.float32) + b2_ref[...]
    qn = qp / jnp.maximum(
        jnp.sqrt(jnp.sum(qp * qp, axis=1, keepdims=True)), 1e-12)

    sim = lax.dot_general(qn, kn_ref[...], (((1,), (1,)), ((), ())),
                          preferred_element_type=jnp.float32)
    sim_ref[...] = sim

    neg = jnp.float32(-jnp.inf)
    iota = lax.broadcasted_iota(jnp.int32, (BN, DICT), 1)
    d = jnp.max(sim, axis=1, keepdims=True)
    d1 = d
    ds = [d]
    for _ in range(K - 1):
        d = jnp.max(jnp.where(sim < d, sim, neg), axis=1, keepdims=True)
        ds.append(d)

    idx_cols = []
    e_cols = []
    z = jnp.zeros((BN, 1), jnp.float32)
    for dj in ds:
        fin = dj > neg
        ij = jnp.min(jnp.where(sim == dj, iota, DICT), axis=1, keepdims=True)
        ej = jnp.where(fin, jnp.exp((dj - d1) * (1.0 / TEMP)), 0.0)
        idx_cols.append(jnp.where(fin, ij, 0))
        e_cols.append(ej)
        z = z + ej
    zinv = 1.0 / z
    pad_i = jnp.zeros((BN, KPAD - K), jnp.int32)
    pad_f = jnp.zeros((BN, KPAD - K), jnp.float32)
    idx_ref[...] = jnp.concatenate(idx_cols + [pad_i], axis=1)
    w_ref[...] = jnp.concatenate([e * zinv for e in e_cols] + [pad_f], axis=1)


def _tc_stage(q_feats, W1, b1, W2, b2, keys):
    kn = pl.pallas_call(
        _norm_kernel,
        out_shape=jax.ShapeDtypeStruct((DICT, KDIM), jnp.float32),
    )(keys)
    grid = (NQ // BN,)
    return pl.pallas_call(
        _block_kernel,
        grid=grid,
        in_specs=[
            pl.BlockSpec((BN, FEAT), lambda i: (i, 0)),
            pl.BlockSpec((FEAT, KDIM), lambda i: (0, 0)),
            pl.BlockSpec((1, KDIM), lambda i: (0, 0)),
            pl.BlockSpec((KDIM, KDIM), lambda i: (0, 0)),
            pl.BlockSpec((1, KDIM), lambda i: (0, 0)),
            pl.BlockSpec((DICT, KDIM), lambda i: (0, 0)),
        ],
        out_specs=[
            pl.BlockSpec((BN, DICT), lambda i: (i, 0)),
            pl.BlockSpec((BN, KPAD), lambda i: (i, 0)),
            pl.BlockSpec((BN, KPAD), lambda i: (i, 0)),
        ],
        out_shape=[
            jax.ShapeDtypeStruct((NQ, DICT), jnp.float32),
            jax.ShapeDtypeStruct((NQ, KPAD), jnp.int32),
            jax.ShapeDtypeStruct((NQ, KPAD), jnp.float32),
        ],
        compiler_params=pltpu.CompilerParams(
            dimension_semantics=("arbitrary",)),
    )(q_feats, W1, b1.reshape(1, KDIM), W2, b2.reshape(1, KDIM), kn)


def _sc_combine_body(values_hbm, idxt_hbm, w_hbm, out_hbm,
                     ic0, ic1, ic2, ic3, ic4, wv,
                     rows0, rows1, rows2, rows3, rows4, acc, sem):
    nc = 2
    wid = lax.axis_index("s") * nc + lax.axis_index("c")
    base = wid * QPW
    rows = [rows0, rows1, rows2, rows3, rows4]
    icols = [ic0, ic1, ic2, ic3, ic4]

    def chunk_body(c, _):
        qbase = base + c * CH
        pltpu.sync_copy(w_hbm.at[pl.ds(qbase, CH)], wv)
        for k in range(K):
            pltpu.sync_copy(idxt_hbm.at[k, pl.ds(qbase, CH)], icols[k])
        copies = []
        for k in range(K):
            copies.append(
                pltpu.async_copy(values_hbm.at[icols[k]], rows[k], sem))
        for cp in copies:
            cp.wait()

        def q_body(q, _):
            w_row = wv[q, :]
            w0 = w_row[0]
            w1 = w_row[1]
            w2 = w_row[2]
            w3 = w_row[3]
            w4 = w_row[4]

            def d_body(d, _):
                b = d * 16
                r = (w0 * rows0[q, pl.ds(b, 16)]
                     + w1 * rows1[q, pl.ds(b, 16)]
                     + w2 * rows2[q, pl.ds(b, 16)]
                     + w3 * rows3[q, pl.ds(b, 16)]
                     + w4 * rows4[q, pl.ds(b, 16)])
                acc[q, pl.ds(b, 16)] = r
                return 0

            lax.fori_loop(0, VDIM // 16, d_body, 0, unroll=4)
            return 0

        lax.fori_loop(0, CH, q_body, 0)
        pltpu.sync_copy(acc, out_hbm.at[pl.ds(qbase, CH)])
        return 0

    lax.fori_loop(0, NCHUNK, chunk_body, 0)


def _sc_stage(values, idx_t, w):
    mesh = plsc.VectorSubcoreMesh(core_axis_name="c", subcore_axis_name="s")
    kfn = functools.partial(
        pl.kernel,
        mesh=mesh,
        out_type=jax.ShapeDtypeStruct((NQ, VDIM), jnp.float32),
        scratch_types=[pltpu.VMEM((CH,), jnp.int32) for _ in range(K)] + [
            pltpu.VMEM((CH, KPAD), jnp.float32),
        ] + [pltpu.VMEM((CH, VDIM), jnp.float32) for _ in range(K)] + [
            pltpu.VMEM((CH, VDIM), jnp.float32),
            pltpu.SemaphoreType.DMA,
        ],
    )(_sc_combine_body)
    return kfn(values, idx_t, w)


@jax.jit
def kernel(q_feats, W1, b1, W2, b2, keys, values, topk):
    del topk
    sim, idx, w = _tc_stage(q_feats, W1, b1, W2, b2, keys)
    idx_t = jnp.transpose(idx)
    retrieved = _sc_stage(values, idx_t, w)
    return retrieved, sim


# speedup vs baseline: 2.3018x; 2.3018x over previous
import jax
import jax.numpy as jnp
from jax import lax
from jax.experimental import pallas as pl
from jax.experimental.pallas import tpu as pltpu

FEAT = 1024
KDIM = 1024
VDIM = 1024
DICT = 2048
NQ = 16384
K = 5
TEMP = 0.15
BN = 512


def _norm_kernel(keys_ref, kn_ref):
    keys = keys_ref[...]
    kn_ref[...] = keys / jnp.maximum(
        jnp.sqrt(jnp.sum(keys * keys, axis=1, keepdims=True)), 1e-12)


def _block_kernel(q_ref, w1_ref, b1_ref, w2_ref, b2_ref, kn_ref, vals_ref,
                  ret_ref, sim_ref):
    x = q_ref[...]
    h = jnp.maximum(
        jnp.dot(x, w1_ref[...], preferred_element_type=jnp.float32)
        + b1_ref[...], 0.0)
    qp = jnp.dot(h, w2_ref[...], preferred_element_type=jnp.float32) + b2_ref[...]
    qn = qp / jnp.maximum(
        jnp.sqrt(jnp.sum(qp * qp, axis=1, keepdims=True)), 1e-12)

    sim = lax.dot_general(qn, kn_ref[...], (((1,), (1,)), ((), ())),
                          preferred_element_type=jnp.float32)
    sim_ref[...] = sim

    neg = jnp.float32(-jnp.inf)
    d = jnp.max(sim, axis=1, keepdims=True)
    d1 = d
    thr = d
    for _ in range(K - 1):
        d = jnp.max(jnp.where(sim < d, sim, neg), axis=1, keepdims=True)
        thr = jnp.where(d > neg, d, thr)
    ew = jnp.exp((sim - d1) * (1.0 / TEMP))
    w = jnp.where(sim >= thr, ew, 0.0)
    z = jnp.sum(w, axis=1, keepdims=True)

    ret = jnp.dot(w, vals_ref[...], preferred_element_type=jnp.float32)
    ret_ref[...] = ret / z


@jax.jit
def kernel(q_feats, W1, b1, W2, b2, keys, values, topk):
    del topk
    kn = pl.pallas_call(
        _norm_kernel,
        out_shape=jax.ShapeDtypeStruct((DICT, KDIM), jnp.float32),
    )(keys)
    grid = (NQ // BN,)
    out = pl.pallas_call(
        _block_kernel,
        grid=grid,
        in_specs=[
            pl.BlockSpec((BN, FEAT), lambda i: (i, 0)),
            pl.BlockSpec((FEAT, KDIM), lambda i: (0, 0)),
            pl.BlockSpec((1, KDIM), lambda i: (0, 0)),
            pl.BlockSpec((KDIM, KDIM), lambda i: (0, 0)),
            pl.BlockSpec((1, KDIM), lambda i: (0, 0)),
            pl.BlockSpec((DICT, KDIM), lambda i: (0, 0)),
            pl.BlockSpec((DICT, VDIM), lambda i: (0, 0)),
        ],
        out_specs=[
            pl.BlockSpec((BN, VDIM), lambda i: (i, 0)),
            pl.BlockSpec((BN, DICT), lambda i: (i, 0)),
        ],
        out_shape=[
            jax.ShapeDtypeStruct((NQ, VDIM), jnp.float32),
            jax.ShapeDtypeStruct((NQ, DICT), jnp.float32),
        ],
        compiler_params=pltpu.CompilerParams(
            dimension_semantics=("arbitrary",)),
    )(q_feats, W1, b1.reshape(1, KDIM), W2, b2.reshape(1, KDIM), kn, values)
    return out[0], out[1]
